# baseline (speedup 1.0000x reference)
import functools

import jax
import jax.numpy as jnp
from jax import lax
from jax.experimental import pallas as pl
from jax.experimental.pallas import tpu as pltpu
from jax.experimental.pallas import tpu_sc as plsc

B = 16384
D = 64
GAMMA = 12.0

NC = 2
NS = 16
NW = NC * NS
BPW = B // NW
CHUNK = 256
NCHUNK = BPW // CHUNK
GROUPS = CHUNK // 16


def _body(hidx_hbm, ridx_hbm, tidx_hbm, ent_hbm, rel_hbm, out_hbm,
          hidx_v, ridx_v, tidx_v, h_v, r_v, t_v, tr_v, out_v,
          sem_h, sem_r, sem_t):
    wid = lax.axis_index("s") * NC + lax.axis_index("c")
    base = wid * BPW

    pltpu.sync_copy(hidx_hbm.at[pl.ds(base, BPW)], hidx_v)
    pltpu.sync_copy(ridx_hbm.at[pl.ds(base, BPW)], ridx_v)
    pltpu.sync_copy(tidx_hbm.at[pl.ds(base, BPW)], tidx_v)

    lanes = lax.iota(jnp.int32, 16)
    tr_idx = lanes * 16

    for chunk in range(NCHUNK):
        co = chunk * CHUNK

        def fetch(g, carry):
            hvec = hidx_v[pl.ds(co + g * 16, 16)]
            rvec = ridx_v[pl.ds(co + g * 16, 16)]
            tvec = tidx_v[pl.ds(co + g * 16, 16)]
            for u in range(16):
                i = g * 16 + u
                pltpu.async_copy(ent_hbm.at[pl.ds(hvec[u], 1), :],
                                 h_v.at[pl.ds(i, 1), :], sem_h)
                pltpu.async_copy(rel_hbm.at[pl.ds(rvec[u], 1), :],
                                 r_v.at[pl.ds(i, 1), :], sem_r)
                pltpu.async_copy(ent_hbm.at[pl.ds(tvec[u], 1), :],
                                 t_v.at[pl.ds(i, 1), :], sem_t)
            return carry

        lax.fori_loop(0, GROUPS, fetch, 0)
        pltpu.make_async_copy(ent_hbm.at[pl.ds(0, CHUNK), :], h_v,
                              sem_h).wait()
        pltpu.make_async_copy(rel_hbm.at[pl.ds(0, CHUNK), :], r_v,
                              sem_r).wait()
        pltpu.make_async_copy(ent_hbm.at[pl.ds(0, CHUNK), :], t_v,
                              sem_t).wait()

        def group(g, carry):
            for u in range(16):
                row = g * 16 + u
                acc = jnp.zeros((16,), jnp.float32)
                for c in range(D // 16):
                    sl = pl.ds(c * 16, 16)
                    acc = acc + jnp.abs(
                        h_v[row, sl] + r_v[row, sl] - t_v[row, sl])
                plsc.store_scatter(tr_v, [tr_idx + u], acc)
            totals = jnp.zeros((16,), jnp.float32)
            for l in range(16):
                totals = totals + tr_v[pl.ds(l * 16, 16)]
            out_v[pl.ds(co + g * 16, 16)] = GAMMA - totals
            return carry

        lax.fori_loop(0, GROUPS, group, 0)

    pltpu.sync_copy(out_v, out_hbm.at[pl.ds(base, BPW)])


@functools.partial(
    pl.kernel,
    out_type=jax.ShapeDtypeStruct((B,), jnp.float32),
    mesh=plsc.VectorSubcoreMesh(core_axis_name="c", subcore_axis_name="s"),
    compiler_params=pltpu.CompilerParams(
        needs_layout_passes=False, use_tc_tiling_on_sc=True),
    scratch_types=[
        pltpu.VMEM((BPW,), jnp.int32),
        pltpu.VMEM((BPW,), jnp.int32),
        pltpu.VMEM((BPW,), jnp.int32),
        pltpu.VMEM((CHUNK, D), jnp.float32),
        pltpu.VMEM((CHUNK, D), jnp.float32),
        pltpu.VMEM((CHUNK, D), jnp.float32),
        pltpu.VMEM((256,), jnp.float32),
        pltpu.VMEM((BPW,), jnp.float32),
        pltpu.SemaphoreType.DMA,
        pltpu.SemaphoreType.DMA,
        pltpu.SemaphoreType.DMA,
    ],
)
def _score_kernel(hidx_hbm, ridx_hbm, tidx_hbm, ent_hbm, rel_hbm, out_hbm,
                  *scratch):
    _body(hidx_hbm, ridx_hbm, tidx_hbm, ent_hbm, rel_hbm, out_hbm, *scratch)


def kernel(sample, entity_embedding, relation_embedding):
    hidx = sample[:, 0].astype(jnp.int32)
    ridx = sample[:, 1].astype(jnp.int32)
    tidx = sample[:, 2].astype(jnp.int32)
    scores = _score_kernel(hidx, ridx, tidx, entity_embedding,
                           relation_embedding)
    return scores[:, None]
